# Initial kernel scaffold; baseline (speedup 1.0000x reference)
import jax
import jax.numpy as jnp
from jax.experimental import pallas as pl


def kernel(x, edge_index, batch, W1, b1, W2, b2, W3, b3, Wc, bc):
    raise NotImplementedError("write your pallas kernel here")


# speedup vs baseline: 6.0361x; 6.0361x over previous
import functools

import jax
import jax.numpy as jnp
from jax import lax
from jax.experimental import pallas as pl
from jax.experimental.pallas import tpu as pltpu
from jax.experimental.pallas import tpu_sc as plsc

_NC = 2
_NS = 16
_NW = _NC * _NS
_CH = 128
_G = 64
_ZCH = 80


def _row_split(n):
  zr = -(-n // _NS)
  zr = -(-zr // _ZCH) * _ZCH
  return zr, zr // _ZCH


@functools.lru_cache(maxsize=None)
def _make_sc_agg(n, e, h):
  ept = e // _NW
  assert ept * _NW == e and ept % 8 == 0
  nfull = ept // _CH
  tail = ept - nfull * _CH
  assert tail % 8 == 0
  assert n % _ZCH == 0
  zr, nchunk = _row_split(n)

  scratch = [
      pltpu.VMEM((_CH,), jnp.int32),
      pltpu.VMEM((_CH,), jnp.int32),
      pltpu.VMEM((_CH, h), jnp.float32),
      pltpu.VMEM((_ZCH, h), jnp.float32),
      pltpu.VMEM_SHARED((n, h), jnp.float32),
      pltpu.SemaphoreType.DMA,
  ]
  if tail:
    scratch += [
        pltpu.VMEM((tail,), jnp.int32),
        pltpu.VMEM((tail,), jnp.int32),
        pltpu.VMEM((tail, h), jnp.float32),
    ]

  mesh = plsc.VectorSubcoreMesh(core_axis_name="c", subcore_axis_name="s",
                                num_cores=_NC, num_subcores=_NS)

  @functools.partial(
      pl.kernel, mesh=mesh,
      out_type=jax.ShapeDtypeStruct((_NC * n, h), jnp.float32),
      scratch_types=scratch)
  def agg(h_hbm, src_hbm, dst_hbm, z_hbm, out_ref, *rest):
    it = iter(rest)
    srcv, dstv, rows, stage, acc, sem = (next(it) for _ in range(6))
    if tail:
      tsrcv, tdstv, trows = (next(it) for _ in range(3))

    c = lax.axis_index("c")
    s = lax.axis_index("s")
    wid = s * _NC + c
    r0 = s * zr

    pltpu.sync_copy(z_hbm, stage)
    for i in range(nchunk):
      r = r0 + i * _ZCH

      @pl.when(r + _ZCH <= n)
      def _():
        pltpu.sync_copy(stage, acc.at[pl.ds(r, _ZCH)])

    plsc.subcore_barrier()

    base = wid * ept

    def step(i, carry):
      off = base + i * _CH
      pltpu.sync_copy(src_hbm.at[pl.ds(off, _CH)], srcv)
      pltpu.sync_copy(dst_hbm.at[pl.ds(off, _CH)], dstv)
      pltpu.async_copy(h_hbm.at[srcv], rows, sem).wait()
      pltpu.sync_copy(rows, acc.at[dstv], add=True)
      return carry

    lax.fori_loop(0, nfull, step, 0)

    if tail:
      off = base + nfull * _CH
      pltpu.sync_copy(src_hbm.at[pl.ds(off, tail)], tsrcv)
      pltpu.sync_copy(dst_hbm.at[pl.ds(off, tail)], tdstv)
      pltpu.async_copy(h_hbm.at[tsrcv], trows, sem).wait()
      pltpu.sync_copy(trows, acc.at[tdstv], add=True)

    plsc.subcore_barrier()

    for i in range(nchunk):
      r = r0 + i * _ZCH

      @pl.when(r + _ZCH <= n)
      def _():
        pltpu.sync_copy(acc.at[pl.ds(r, _ZCH)], stage)
        pltpu.sync_copy(stage, out_ref.at[pl.ds(c * n + r, _ZCH)])

  return agg


@functools.lru_cache(maxsize=None)
def _make_sc_deg(n, e, h):
  ept = e // _NW
  nfull = ept // _CH
  tail = ept - nfull * _CH
  assert n % _ZCH == 0
  zr, nchunk = _row_split(n)

  scratch = [
      pltpu.VMEM((_CH,), jnp.int32),
      pltpu.VMEM((_CH, h), jnp.float32),
      pltpu.VMEM((_ZCH, h), jnp.float32),
      pltpu.VMEM_SHARED((n, h), jnp.float32),
  ]
  if tail:
    scratch += [
        pltpu.VMEM((tail,), jnp.int32),
        pltpu.VMEM((tail, h), jnp.float32),
    ]

  mesh = plsc.VectorSubcoreMesh(core_axis_name="c", subcore_axis_name="s",
                                num_cores=_NC, num_subcores=_NS)

  @functools.partial(
      pl.kernel, mesh=mesh,
      out_type=jax.ShapeDtypeStruct((_NC * n, h), jnp.float32),
      scratch_types=scratch)
  def deg(dst_hbm, ones_hbm, zd_hbm, out_ref, *rest):
    it = iter(rest)
    dstv, onesv, dstage, dacc = (next(it) for _ in range(4))
    if tail:
      tdstv, tonesv = (next(it) for _ in range(2))

    c = lax.axis_index("c")
    s = lax.axis_index("s")
    wid = s * _NC + c
    r0 = s * zr

    pltpu.sync_copy(ones_hbm, onesv)
    if tail:
      pltpu.sync_copy(ones_hbm.at[pl.ds(0, tail)], tonesv)
    pltpu.sync_copy(zd_hbm, dstage)
    for i in range(nchunk):
      r = r0 + i * _ZCH

      @pl.when(r + _ZCH <= n)
      def _():
        pltpu.sync_copy(dstage, dacc.at[pl.ds(r, _ZCH)])

    plsc.subcore_barrier()

    base = wid * ept

    def step(i, carry):
      off = base + i * _CH
      pltpu.sync_copy(dst_hbm.at[pl.ds(off, _CH)], dstv)
      pltpu.sync_copy(onesv, dacc.at[dstv], add=True)
      return carry

    lax.fori_loop(0, nfull, step, 0)

    if tail:
      off = base + nfull * _CH
      pltpu.sync_copy(dst_hbm.at[pl.ds(off, tail)], tdstv)
      pltpu.sync_copy(tonesv, dacc.at[tdstv], add=True)

    plsc.subcore_barrier()

    for i in range(nchunk):
      r = r0 + i * _ZCH

      @pl.when(r + _ZCH <= n)
      def _():
        pltpu.sync_copy(dacc.at[pl.ds(r, _ZCH)], dstage)
        pltpu.sync_copy(dstage, out_ref.at[pl.ds(c * n + r, _ZCH)])

  return deg


def _tc_layer(agg_parts, deg_parts, w, b2d):
  n, h = agg_parts.shape[1], agg_parts.shape[2]
  blk = 2000 if n % 2000 == 0 else n
  grid = n // blk

  def body(ap, dp, wr, br, o):
    a = ap[0] + ap[1]
    deg = dp[0] + dp[1]
    dinv = 1.0 / jnp.maximum(deg, 1.0)
    z = lax.dot_general(a * dinv, wr[...], (((1,), (0,)), ((), ())),
                        preferred_element_type=jnp.float32)
    o[...] = jnp.maximum(z + br[...], 0.0)

  return pl.pallas_call(
      body,
      grid=(grid,),
      in_specs=[
          pl.BlockSpec((2, blk, h), lambda i: (0, i, 0)),
          pl.BlockSpec((2, blk, 1), lambda i: (0, i, 0)),
          pl.BlockSpec((h, h), lambda i: (0, 0)),
          pl.BlockSpec((1, h), lambda i: (0, 0)),
      ],
      out_specs=pl.BlockSpec((blk, h), lambda i: (i, 0)),
      out_shape=jax.ShapeDtypeStruct((n, h), jnp.float32),
  )(agg_parts, deg_parts, w, b2d)


def _tc_pool(h3, batch2d, wc, bc2d):
  n, h = h3.shape

  def body(hr, br, wr, bcr, o):
    gids = lax.broadcasted_iota(jnp.int32, (_G, n), 0)
    m = (br[...] == gids).astype(jnp.float32)
    counts = jnp.sum(m, axis=1, keepdims=True)
    pooled = lax.dot_general(m, hr[...], (((1,), (0,)), ((), ())),
                             preferred_element_type=jnp.float32,
                             precision=lax.Precision.HIGHEST)
    pooled = pooled / jnp.maximum(counts, 1.0)
    o[...] = lax.dot_general(pooled, wr[...], (((1,), (0,)), ((), ())),
                             preferred_element_type=jnp.float32) + bcr[...]

  return pl.pallas_call(
      body,
      out_shape=jax.ShapeDtypeStruct((_G, 1), jnp.float32),
  )(h3, batch2d, wc, bc2d)


def kernel(x, edge_index, batch, W1, b1, W2, b2, W3, b3, Wc, bc):
  n, d = x.shape
  e = edge_index.shape[1]
  src = edge_index[0]
  dst = edge_index[1]
  z_rows = jnp.zeros((_ZCH, d), jnp.float32)
  ones_ch = jnp.ones((_CH, d), jnp.float32)

  agg = _make_sc_agg(n, e, d)
  deg = _make_sc_deg(n, e, d)

  degp = deg(dst, ones_ch, z_rows)[:, 0:1].reshape(_NC, n, 1)
  a1 = agg(x, src, dst, z_rows)
  h1 = _tc_layer(a1.reshape(_NC, n, d), degp, W1, b1.reshape(1, -1))
  a2 = agg(h1, src, dst, z_rows)
  h2 = _tc_layer(a2.reshape(_NC, n, d), degp, W2, b2.reshape(1, -1))
  a3 = agg(h2, src, dst, z_rows)
  h3 = _tc_layer(a3.reshape(_NC, n, d), degp, W3, b3.reshape(1, -1))
  return _tc_pool(h3, batch.reshape(1, -1), Wc, bc.reshape(1, 1))


# speedup vs baseline: 11.3122x; 1.8741x over previous
import functools

import jax
import jax.numpy as jnp
from jax import lax
from jax.experimental import pallas as pl
from jax.experimental.pallas import tpu as pltpu
from jax.experimental.pallas import tpu_sc as plsc

_NC = 2
_NS = 16
_NW = _NC * _NS
_CH = 128
_G = 64
_ZCH = 40


def _row_split(n):
  zr = -(-n // _NS)
  zr = -(-zr // _ZCH) * _ZCH
  return zr, zr // _ZCH


def _tile_windows(nwin):
  per = nwin // _NW
  extra = nwin - per * _NW
  return per, extra


def _zero_acc(z_hbm, stage, acc, s, n, zr, nchunk):
  pltpu.sync_copy(z_hbm, stage)
  r0 = s * zr
  for i in range(nchunk):
    r = r0 + i * _ZCH

    @pl.when(r + _ZCH <= n)
    def _():
      pltpu.sync_copy(stage, acc.at[pl.ds(r, _ZCH)])


def _export_acc(out_ref, stage, acc, c, s, n, zr, nchunk):
  r0 = s * zr
  for i in range(nchunk):
    r = r0 + i * _ZCH

    @pl.when(r + _ZCH <= n)
    def _():
      pltpu.sync_copy(acc.at[pl.ds(r, _ZCH)], stage)
      pltpu.sync_copy(stage, out_ref.at[pl.ds(c * n + r, _ZCH)])


@functools.lru_cache(maxsize=None)
def _make_sc_agg(n, e, h):
  assert e % _CH == 0
  nwin = e // _CH
  per, extra = _tile_windows(nwin)
  assert per >= 4
  nsteps = (per - 2) // 4
  rem_base = 2 + nsteps * 4
  assert n % _ZCH == 0
  zr, nchunk = _row_split(n)

  scratch = [
      pltpu.VMEM((2, _CH), jnp.int32),
      pltpu.VMEM((2, _CH), jnp.int32),
      pltpu.VMEM((2, _CH), jnp.int32),
      pltpu.VMEM((2, _CH), jnp.int32),
      pltpu.VMEM((_CH, h), jnp.float32),
      pltpu.VMEM((_CH, h), jnp.float32),
      pltpu.VMEM((_ZCH, h), jnp.float32),
      pltpu.VMEM_SHARED((n, h), jnp.float32),
      pltpu.SemaphoreType.DMA,
      pltpu.SemaphoreType.DMA,
      pltpu.SemaphoreType.DMA,
  ]

  mesh = plsc.VectorSubcoreMesh(core_axis_name="c", subcore_axis_name="s",
                                num_cores=_NC, num_subcores=_NS)

  @functools.partial(
      pl.kernel, mesh=mesh,
      out_type=jax.ShapeDtypeStruct((_NC * n, h), jnp.float32),
      scratch_types=scratch)
  def agg(h_hbm, src_hbm, dst_hbm, z_hbm, out_ref,
          sca, dca, scb, dcb, rowsa, rowsb, stage, acc, sga, sgb, sidx):
    c = lax.axis_index("c")
    s = lax.axis_index("s")
    wid = s * _NC + c
    _zero_acc(z_hbm, stage, acc, s, n, zr, nchunk)
    plsc.subcore_barrier()

    base = wid * per + jnp.minimum(wid, extra)
    nw = per + jnp.where(wid < extra, 1, 0)
    hi = nwin - 2

    def ldpair(w, sv, dv):
      wc = jnp.minimum(w, hi) * _CH
      a = pltpu.async_copy(src_hbm.at[pl.ds(wc, _CH)], sv.at[0], sidx)
      a2 = pltpu.async_copy(src_hbm.at[pl.ds(wc + _CH, _CH)], sv.at[1], sidx)
      b = pltpu.async_copy(dst_hbm.at[pl.ds(wc, _CH)], dv.at[0], sidx)
      b2 = pltpu.async_copy(dst_hbm.at[pl.ds(wc + _CH, _CH)], dv.at[1], sidx)
      return (a, a2, b, b2)

    def gather(sv, j, rows, sem):
      return pltpu.async_copy(h_hbm.at[sv.at[j]], rows, sem)

    def scat(rows, dv, j):
      pltpu.sync_copy(rows, acc.at[dv.at[j]], add=True)

    for hnd in ldpair(base, sca, dca):
      hnd.wait()
    gather(sca, 0, rowsa, sga)
    gather(sca, 1, rowsb, sgb)
    for hnd in ldpair(base + 2, scb, dcb):
      hnd.wait()

    def wait_g(rows, sem):
      pltpu.make_async_copy(h_hbm.at[pl.ds(0, _CH)], rows, sem).wait()

    def step(i, carry):
      w = base + i * 4
      wait_g(rowsa, sga)
      scat(rowsa, dca, 0)
      gather(scb, 0, rowsa, sga)
      wait_g(rowsb, sgb)
      scat(rowsb, dca, 1)
      gather(scb, 1, rowsb, sgb)
      hnds = ldpair(w + 4, sca, dca)
      wait_g(rowsa, sga)
      scat(rowsa, dcb, 0)
      for hnd in hnds:
        hnd.wait()
      gather(sca, 0, rowsa, sga)
      wait_g(rowsb, sgb)
      scat(rowsb, dcb, 1)
      gather(sca, 1, rowsb, sgb)
      for hnd in ldpair(w + 6, scb, dcb):
        hnd.wait()
      return carry

    lax.fori_loop(0, nsteps, step, 0)

    wait_g(rowsa, sga)
    scat(rowsa, dca, 0)
    wait_g(rowsb, sgb)
    scat(rowsb, dca, 1)
    for j in range(4):
      @pl.when(rem_base + j < nw)
      def _():
        w = (base + rem_base + j) * _CH
        a = pltpu.async_copy(src_hbm.at[pl.ds(w, _CH)], sca.at[0], sidx)
        b = pltpu.async_copy(dst_hbm.at[pl.ds(w, _CH)], dca.at[0], sidx)
        a.wait()
        b.wait()
        gather(sca, 0, rowsa, sga)
        wait_g(rowsa, sga)
        scat(rowsa, dca, 0)

    plsc.subcore_barrier()
    _export_acc(out_ref, stage, acc, c, s, n, zr, nchunk)

  return agg


@functools.lru_cache(maxsize=None)
def _make_sc_deg(n, e, h):
  assert e % _CH == 0
  nwin = e // _CH
  per, extra = _tile_windows(nwin)
  assert n % _ZCH == 0
  zr, nchunk = _row_split(n)

  scratch = [
      pltpu.VMEM((2, _CH), jnp.int32),
      pltpu.VMEM((2, _CH), jnp.int32),
      pltpu.VMEM((_CH, h), jnp.float32),
      pltpu.VMEM((_ZCH, h), jnp.float32),
      pltpu.VMEM_SHARED((n, h), jnp.float32),
      pltpu.SemaphoreType.DMA,
  ]

  mesh = plsc.VectorSubcoreMesh(core_axis_name="c", subcore_axis_name="s",
                                num_cores=_NC, num_subcores=_NS)

  @functools.partial(
      pl.kernel, mesh=mesh,
      out_type=jax.ShapeDtypeStruct((_NC * n, h), jnp.float32),
      scratch_types=scratch)
  def deg(dst_hbm, ones_hbm, z_hbm, out_ref, dca, dcb, onesv, stage, acc, sidx):
    c = lax.axis_index("c")
    s = lax.axis_index("s")
    wid = s * _NC + c
    pltpu.sync_copy(ones_hbm, onesv)
    _zero_acc(z_hbm, stage, acc, s, n, zr, nchunk)
    plsc.subcore_barrier()

    base = wid * per + jnp.minimum(wid, extra)
    nw = per + jnp.where(wid < extra, 1, 0)
    hi = nwin - 2
    nsteps = per // 4
    rem_base = nsteps * 4

    def scat(dv, j):
      pltpu.sync_copy(onesv, acc.at[dv.at[j]], add=True)

    def lddpair(w, dv):
      wc = jnp.minimum(w, hi) * _CH
      b = pltpu.async_copy(dst_hbm.at[pl.ds(wc, _CH)], dv.at[0], sidx)
      b2 = pltpu.async_copy(dst_hbm.at[pl.ds(wc + _CH, _CH)], dv.at[1], sidx)
      return (b, b2)

    for hnd in lddpair(base, dca):
      hnd.wait()

    def step(i, carry):
      w = base + i * 4
      hnds = lddpair(w + 2, dcb)
      scat(dca, 0)
      scat(dca, 1)
      for hnd in hnds:
        hnd.wait()
      hnds = lddpair(w + 4, dca)
      scat(dcb, 0)
      scat(dcb, 1)
      for hnd in hnds:
        hnd.wait()
      return carry

    lax.fori_loop(0, nsteps, step, 0)

    for j in range(4):
      @pl.when(rem_base + j < nw)
      def _():
        ia3 = pltpu.async_copy(dst_hbm.at[pl.ds((base + rem_base + j) * _CH, _CH)],
                               dca.at[0], sidx)
        ia3.wait()
        scat(dca, 0)

    plsc.subcore_barrier()
    _export_acc(out_ref, stage, acc, c, s, n, zr, nchunk)

  return deg


def _tc_layer(agg_parts, deg_parts, w, b2d):
  n, h = agg_parts.shape[1], agg_parts.shape[2]
  blk = 2000 if n % 2000 == 0 else n
  grid = n // blk

  def body(ap, dp, wr, br, o):
    a = ap[0] + ap[1]
    deg = dp[0] + dp[1]
    dinv = 1.0 / jnp.maximum(deg, 1.0)
    z = lax.dot_general(a * dinv, wr[...], (((1,), (0,)), ((), ())),
                        preferred_element_type=jnp.float32)
    o[...] = jnp.maximum(z + br[...], 0.0)

  return pl.pallas_call(
      body,
      grid=(grid,),
      in_specs=[
          pl.BlockSpec((2, blk, h), lambda i: (0, i, 0)),
          pl.BlockSpec((2, blk, 1), lambda i: (0, i, 0)),
          pl.BlockSpec((h, h), lambda i: (0, 0)),
          pl.BlockSpec((1, h), lambda i: (0, 0)),
      ],
      out_specs=pl.BlockSpec((blk, h), lambda i: (i, 0)),
      out_shape=jax.ShapeDtypeStruct((n, h), jnp.float32),
  )(agg_parts, deg_parts, w, b2d)


def _tc_pool(h3, batch2d, wc, bc2d):
  n, h = h3.shape

  def body(hr, br, wr, bcr, o):
    gids = lax.broadcasted_iota(jnp.int32, (_G, n), 0)
    m = (br[...] == gids).astype(jnp.float32)
    counts = jnp.sum(m, axis=1, keepdims=True)
    pooled = lax.dot_general(m, hr[...], (((1,), (0,)), ((), ())),
                             preferred_element_type=jnp.float32,
                             precision=lax.Precision.HIGHEST)
    pooled = pooled / jnp.maximum(counts, 1.0)
    o[...] = lax.dot_general(pooled, wr[...], (((1,), (0,)), ((), ())),
                             preferred_element_type=jnp.float32) + bcr[...]

  return pl.pallas_call(
      body,
      out_shape=jax.ShapeDtypeStruct((_G, 1), jnp.float32),
  )(h3, batch2d, wc, bc2d)


def kernel(x, edge_index, batch, W1, b1, W2, b2, W3, b3, Wc, bc):
  n, d = x.shape
  e = edge_index.shape[1]
  src = edge_index[0]
  dst = edge_index[1]
  z_rows = jnp.zeros((_ZCH, d), jnp.float32)
  ones_ch = jnp.ones((_CH, d), jnp.float32)

  agg = _make_sc_agg(n, e, d)
  deg = _make_sc_deg(n, e, d)

  degp = deg(dst, ones_ch, z_rows)[:, 0:1].reshape(_NC, n, 1)
  a1 = agg(x, src, dst, z_rows)
  h1 = _tc_layer(a1.reshape(_NC, n, d), degp, W1, b1.reshape(1, -1))
  a2 = agg(h1, src, dst, z_rows)
  h2 = _tc_layer(a2.reshape(_NC, n, d), degp, W2, b2.reshape(1, -1))
  a3 = agg(h2, src, dst, z_rows)
  h3 = _tc_layer(a3.reshape(_NC, n, d), degp, W3, b3.reshape(1, -1))
  return _tc_pool(h3, batch.reshape(1, -1), Wc, bc.reshape(1, 1))


# speedup vs baseline: 13.1547x; 1.1629x over previous
import functools

import jax
import jax.numpy as jnp
from jax import lax
from jax.experimental import pallas as pl
from jax.experimental.pallas import tpu as pltpu
from jax.experimental.pallas import tpu_sc as plsc

_NC = 2
_NS = 16
_NW = _NC * _NS
_CH = 128
_G = 64
_ZCH = 40


def _row_split(n):
  zr = -(-n // _NS)
  zr = -(-zr // _ZCH) * _ZCH
  return zr, zr // _ZCH


def _tile_windows(nwin):
  per = nwin // _NW
  extra = nwin - per * _NW
  return per, extra


def _zero_acc(z_hbm, stage, acc, s, n, zr, nchunk):
  pltpu.sync_copy(z_hbm, stage)
  r0 = s * zr
  for i in range(nchunk):
    r = r0 + i * _ZCH

    @pl.when(r + _ZCH <= n)
    def _():
      pltpu.sync_copy(stage, acc.at[pl.ds(r, _ZCH)])


def _export_acc(out_ref, stage, acc, c, s, n, zr, nchunk):
  r0 = s * zr
  for i in range(nchunk):
    r = r0 + i * _ZCH

    @pl.when(r + _ZCH <= n)
    def _():
      pltpu.sync_copy(acc.at[pl.ds(r, _ZCH)], stage)
      pltpu.sync_copy(stage, out_ref.at[pl.ds(c * n + r, _ZCH)])


@functools.lru_cache(maxsize=None)
def _make_sc_agg(n, e, h):
  assert e % _CH == 0
  nwin = e // _CH
  per, extra = _tile_windows(nwin)
  assert per >= 4
  nsteps = (per - 2) // 4
  rem_base = 2 + nsteps * 4
  assert n % _ZCH == 0
  zr, nchunk = _row_split(n)

  scratch = [
      pltpu.VMEM((2, _CH), jnp.int32),
      pltpu.VMEM((2, _CH), jnp.int32),
      pltpu.VMEM((2, _CH), jnp.int32),
      pltpu.VMEM((2, _CH), jnp.int32),
      pltpu.VMEM((_CH, h), jnp.float32),
      pltpu.VMEM((_CH, h), jnp.float32),
      pltpu.VMEM((_ZCH, h), jnp.float32),
      pltpu.VMEM_SHARED((n, h), jnp.float32),
      pltpu.SemaphoreType.DMA,
      pltpu.SemaphoreType.DMA,
      pltpu.SemaphoreType.DMA,
  ]

  mesh = plsc.VectorSubcoreMesh(core_axis_name="c", subcore_axis_name="s",
                                num_cores=_NC, num_subcores=_NS)

  @functools.partial(
      pl.kernel, mesh=mesh,
      out_type=jax.ShapeDtypeStruct((_NC * n, h), jnp.float32),
      scratch_types=scratch)
  def agg(h_hbm, src_hbm, dst_hbm, z_hbm, out_ref,
          sca, dca, scb, dcb, rowsa, rowsb, stage, acc, sga, sgb, sidx):
    c = lax.axis_index("c")
    s = lax.axis_index("s")
    wid = s * _NC + c
    _zero_acc(z_hbm, stage, acc, s, n, zr, nchunk)
    plsc.subcore_barrier()

    base = wid * per + jnp.minimum(wid, extra)
    nw = per + jnp.where(wid < extra, 1, 0)
    hi = nwin - 2

    def ldpair(w, sv, dv):
      wc = jnp.minimum(w, hi) * _CH
      a = pltpu.async_copy(src_hbm.at[pl.ds(wc, _CH)], sv.at[0], sidx)
      a2 = pltpu.async_copy(src_hbm.at[pl.ds(wc + _CH, _CH)], sv.at[1], sidx)
      b = pltpu.async_copy(dst_hbm.at[pl.ds(wc, _CH)], dv.at[0], sidx)
      b2 = pltpu.async_copy(dst_hbm.at[pl.ds(wc + _CH, _CH)], dv.at[1], sidx)
      return (a, a2, b, b2)

    def gather(sv, j, rows, sem):
      return pltpu.async_copy(h_hbm.at[sv.at[j]], rows, sem)

    def scat(rows, dv, j):
      pltpu.sync_copy(rows, acc.at[dv.at[j]], add=True)

    for hnd in ldpair(base, sca, dca):
      hnd.wait()
    gather(sca, 0, rowsa, sga)
    gather(sca, 1, rowsb, sgb)
    for hnd in ldpair(base + 2, scb, dcb):
      hnd.wait()

    def wait_g(rows, sem):
      pltpu.make_async_copy(h_hbm.at[pl.ds(0, _CH)], rows, sem).wait()

    def step(i, carry):
      w = base + i * 4
      wait_g(rowsa, sga)
      scat(rowsa, dca, 0)
      gather(scb, 0, rowsa, sga)
      wait_g(rowsb, sgb)
      scat(rowsb, dca, 1)
      gather(scb, 1, rowsb, sgb)
      hnds = ldpair(w + 4, sca, dca)
      wait_g(rowsa, sga)
      scat(rowsa, dcb, 0)
      for hnd in hnds:
        hnd.wait()
      gather(sca, 0, rowsa, sga)
      wait_g(rowsb, sgb)
      scat(rowsb, dcb, 1)
      gather(sca, 1, rowsb, sgb)
      for hnd in ldpair(w + 6, scb, dcb):
        hnd.wait()
      return carry

    lax.fori_loop(0, nsteps, step, 0)

    wait_g(rowsa, sga)
    scat(rowsa, dca, 0)
    wait_g(rowsb, sgb)
    scat(rowsb, dca, 1)
    for j in range(4):
      @pl.when(rem_base + j < nw)
      def _():
        w = (base + rem_base + j) * _CH
        a = pltpu.async_copy(src_hbm.at[pl.ds(w, _CH)], sca.at[0], sidx)
        b = pltpu.async_copy(dst_hbm.at[pl.ds(w, _CH)], dca.at[0], sidx)
        a.wait()
        b.wait()
        gather(sca, 0, rowsa, sga)
        wait_g(rowsa, sga)
        scat(rowsa, dca, 0)

    plsc.subcore_barrier()
    _export_acc(out_ref, stage, acc, c, s, n, zr, nchunk)

  return agg


@functools.lru_cache(maxsize=None)
def _make_sc_deg(n, e):
  assert e % _CH == 0
  nwin = e // _CH
  per, extra = _tile_windows(nwin)
  assert n % 16 == 0

  scratch = [
      pltpu.VMEM((per * _CH,), jnp.int32),
      pltpu.VMEM((_CH,), jnp.int32),
      pltpu.VMEM((n,), jnp.float32),
      pltpu.SemaphoreType.DMA,
  ]
  mesh = plsc.VectorSubcoreMesh(core_axis_name="c", subcore_axis_name="s",
                                num_cores=_NC, num_subcores=_NS)

  @functools.partial(
      pl.kernel, mesh=mesh,
      out_type=jax.ShapeDtypeStruct((_NW * n,), jnp.float32),
      scratch_types=scratch,
      compiler_params=pltpu.CompilerParams(needs_layout_passes=False))
  def deg(dst_hbm, out_ref, dall, dx, cnt, sem):
    c = lax.axis_index("c")
    s = lax.axis_index("s")
    wid = s * _NC + c
    base = wid * per + jnp.minimum(wid, extra)
    nw = per + jnp.where(wid < extra, 1, 0)

    def zstep(i, carry):
      cnt[pl.ds(i * 16, 16)] = jnp.zeros((16,), jnp.float32)
      return carry

    lax.fori_loop(0, n // 16, zstep, 0)

    pltpu.sync_copy(dst_hbm.at[pl.ds(base * _CH, per * _CH)], dall)

    @pl.when(nw > per)
    def _():
      pltpu.sync_copy(dst_hbm.at[pl.ds((base + per) * _CH, _CH)], dx)

    ones16 = jnp.ones((16,), jnp.float32)

    def hstep(i, carry):
      plsc.addupdate_scatter(cnt, [dall[pl.ds(i * 16, 16)]], ones16)
      return carry

    lax.fori_loop(0, per * _CH // 16, hstep, 0)

    @pl.when(nw > per)
    def _():
      def hstep2(i, carry):
        plsc.addupdate_scatter(cnt, [dx[pl.ds(i * 16, 16)]], ones16)
        return carry

      lax.fori_loop(0, _CH // 16, hstep2, 0)

    pltpu.sync_copy(cnt, out_ref.at[pl.ds(wid * n, n)])

  return deg


def _tc_dinv(cnt2d):
  n = cnt2d.shape[1]

  def body(cr, o):
    s = jnp.sum(cr[...], axis=0)
    o[...] = (1.0 / jnp.maximum(s, 1.0))[:, None]

  return pl.pallas_call(
      body,
      out_shape=jax.ShapeDtypeStruct((n, 1), jnp.float32),
  )(cnt2d)


def _tc_layer(agg_parts, dinv, w, b2d):
  n, h = agg_parts.shape[1], agg_parts.shape[2]
  blk = 2000 if n % 2000 == 0 else n
  grid = n // blk

  def body(ap, dp, wr, br, o):
    a = ap[0] + ap[1]
    z = lax.dot_general(a * dp[...], wr[...], (((1,), (0,)), ((), ())),
                        preferred_element_type=jnp.float32)
    o[...] = jnp.maximum(z + br[...], 0.0)

  return pl.pallas_call(
      body,
      grid=(grid,),
      in_specs=[
          pl.BlockSpec((2, blk, h), lambda i: (0, i, 0)),
          pl.BlockSpec((blk, 1), lambda i: (i, 0)),
          pl.BlockSpec((h, h), lambda i: (0, 0)),
          pl.BlockSpec((1, h), lambda i: (0, 0)),
      ],
      out_specs=pl.BlockSpec((blk, h), lambda i: (i, 0)),
      out_shape=jax.ShapeDtypeStruct((n, h), jnp.float32),
  )(agg_parts, dinv, w, b2d)


def _tc_pool(h3, batch2d, wc, bc2d):
  n, h = h3.shape

  def body(hr, br, wr, bcr, o):
    gids = lax.broadcasted_iota(jnp.int32, (_G, n), 0)
    m = (br[...] == gids).astype(jnp.float32)
    counts = jnp.sum(m, axis=1, keepdims=True)
    pooled = lax.dot_general(m, hr[...], (((1,), (0,)), ((), ())),
                             preferred_element_type=jnp.float32,
                             precision=lax.Precision.HIGHEST)
    pooled = pooled / jnp.maximum(counts, 1.0)
    o[...] = lax.dot_general(pooled, wr[...], (((1,), (0,)), ((), ())),
                             preferred_element_type=jnp.float32) + bcr[...]

  return pl.pallas_call(
      body,
      out_shape=jax.ShapeDtypeStruct((_G, 1), jnp.float32),
  )(h3, batch2d, wc, bc2d)


def kernel(x, edge_index, batch, W1, b1, W2, b2, W3, b3, Wc, bc):
  n, d = x.shape
  e = edge_index.shape[1]
  src = edge_index[0]
  dst = edge_index[1]
  z_rows = jnp.zeros((_ZCH, d), jnp.float32)

  agg = _make_sc_agg(n, e, d)
  deg = _make_sc_deg(n, e)

  dinv = _tc_dinv(deg(dst).reshape(_NW, n))
  a1 = agg(x, src, dst, z_rows)
  h1 = _tc_layer(a1.reshape(_NC, n, d), dinv, W1, b1.reshape(1, -1))
  a2 = agg(h1, src, dst, z_rows)
  h2 = _tc_layer(a2.reshape(_NC, n, d), dinv, W2, b2.reshape(1, -1))
  a3 = agg(h2, src, dst, z_rows)
  h3 = _tc_layer(a3.reshape(_NC, n, d), dinv, W3, b3.reshape(1, -1))
  return _tc_pool(h3, batch.reshape(1, -1), Wc, bc.reshape(1, 1))


# speedup vs baseline: 13.4625x; 1.0234x over previous
import functools

import jax
import jax.numpy as jnp
from jax import lax
from jax.experimental import pallas as pl
from jax.experimental.pallas import tpu as pltpu
from jax.experimental.pallas import tpu_sc as plsc

_NC = 2
_NS = 16
_NW = _NC * _NS
_CH = 128
_G = 64
_ZCH = 40


def _row_split(n):
  zr = -(-n // _NS)
  zr = -(-zr // _ZCH) * _ZCH
  return zr, zr // _ZCH


def _tile_windows(nwin):
  per = nwin // _NW
  extra = nwin - per * _NW
  return per, extra


def _zero_acc(z_hbm, stage, acc, sem, s, n, zr, nchunk):
  pltpu.sync_copy(z_hbm, stage)
  r0 = s * zr
  for i in range(nchunk):
    r = r0 + i * _ZCH

    @pl.when(r + _ZCH <= n)
    def _():
      pltpu.async_copy(stage, acc.at[pl.ds(r, _ZCH)], sem)
  for i in range(nchunk):
    r = r0 + i * _ZCH

    @pl.when(r + _ZCH <= n)
    def _():
      pltpu.make_async_copy(z_hbm, stage, sem).wait()


def _export_acc(out_ref, stga, stgb, z_hbm, acc, sem_r, sem_w, c, s, n, zr, nchunk):
  r0 = s * zr
  stages = (stga, stgb)
  for i in range(nchunk):
    r = r0 + i * _ZCH
    stg = stages[i % 2]

    @pl.when(r + _ZCH <= n)
    def _():
      if i >= 2:
        pltpu.make_async_copy(z_hbm, stg, sem_w).wait()
      pltpu.async_copy(acc.at[pl.ds(r, _ZCH)], stg, sem_r)
      pltpu.make_async_copy(z_hbm, stg, sem_r).wait()
      pltpu.async_copy(stg, out_ref.at[pl.ds(c * n + r, _ZCH)], sem_w)
  for i in range(nchunk):
    r = r0 + i * _ZCH
    r2 = r0 + (i + 2) * _ZCH

    @pl.when((r + _ZCH <= n) & (r2 + _ZCH > n))
    def _():
      pltpu.make_async_copy(z_hbm, stages[i % 2], sem_w).wait()


@functools.lru_cache(maxsize=None)
def _make_sc_agg(n, e, h):
  assert e % _CH == 0
  nwin = e // _CH
  per, extra = _tile_windows(nwin)
  assert per >= 4
  nsteps = (per - 2) // 4
  rem_base = 2 + nsteps * 4
  assert n % _ZCH == 0
  zr, nchunk = _row_split(n)

  scratch = [
      pltpu.VMEM((2, _CH), jnp.int32),
      pltpu.VMEM((2, _CH), jnp.int32),
      pltpu.VMEM((2, _CH), jnp.int32),
      pltpu.VMEM((2, _CH), jnp.int32),
      pltpu.VMEM((_CH, h), jnp.float32),
      pltpu.VMEM((_CH, h), jnp.float32),
      pltpu.VMEM((_ZCH, h), jnp.float32),
      pltpu.VMEM((_ZCH, h), jnp.float32),
      pltpu.VMEM_SHARED((n, h), jnp.float32),
      pltpu.SemaphoreType.DMA,
      pltpu.SemaphoreType.DMA,
      pltpu.SemaphoreType.DMA,
  ]

  mesh = plsc.VectorSubcoreMesh(core_axis_name="c", subcore_axis_name="s",
                                num_cores=_NC, num_subcores=_NS)

  @functools.partial(
      pl.kernel, mesh=mesh,
      out_type=jax.ShapeDtypeStruct((_NC * n, h), jnp.float32),
      scratch_types=scratch)
  def agg(h_hbm, src_hbm, dst_hbm, z_hbm, out_ref,
          sca, dca, scb, dcb, rowsa, rowsb, stga, stgb, acc, sga, sgb, sidx):
    c = lax.axis_index("c")
    s = lax.axis_index("s")
    wid = s * _NC + c
    _zero_acc(z_hbm, stga, acc, sga, s, n, zr, nchunk)
    plsc.subcore_barrier()

    base = wid * per + jnp.minimum(wid, extra)
    nw = per + jnp.where(wid < extra, 1, 0)
    hi = nwin - 2

    def ldpair(w, sv, dv):
      wc = jnp.minimum(w, hi) * _CH
      a = pltpu.async_copy(src_hbm.at[pl.ds(wc, _CH)], sv.at[0], sidx)
      a2 = pltpu.async_copy(src_hbm.at[pl.ds(wc + _CH, _CH)], sv.at[1], sidx)
      b = pltpu.async_copy(dst_hbm.at[pl.ds(wc, _CH)], dv.at[0], sidx)
      b2 = pltpu.async_copy(dst_hbm.at[pl.ds(wc + _CH, _CH)], dv.at[1], sidx)
      return (a, a2, b, b2)

    def gather(sv, j, rows, sem):
      return pltpu.async_copy(h_hbm.at[sv.at[j]], rows, sem)

    def scat(rows, dv, j):
      pltpu.sync_copy(rows, acc.at[dv.at[j]], add=True)

    for hnd in ldpair(base, sca, dca):
      hnd.wait()
    gather(sca, 0, rowsa, sga)
    gather(sca, 1, rowsb, sgb)
    for hnd in ldpair(base + 2, scb, dcb):
      hnd.wait()

    def wait_g(rows, sem):
      pltpu.make_async_copy(h_hbm.at[pl.ds(0, _CH)], rows, sem).wait()

    def step(i, carry):
      w = base + i * 4
      wait_g(rowsa, sga)
      scat(rowsa, dca, 0)
      gather(scb, 0, rowsa, sga)
      wait_g(rowsb, sgb)
      scat(rowsb, dca, 1)
      gather(scb, 1, rowsb, sgb)
      hnds = ldpair(w + 4, sca, dca)
      wait_g(rowsa, sga)
      scat(rowsa, dcb, 0)
      for hnd in hnds:
        hnd.wait()
      gather(sca, 0, rowsa, sga)
      wait_g(rowsb, sgb)
      scat(rowsb, dcb, 1)
      gather(sca, 1, rowsb, sgb)
      for hnd in ldpair(w + 6, scb, dcb):
        hnd.wait()
      return carry

    lax.fori_loop(0, nsteps, step, 0)

    wait_g(rowsa, sga)
    scat(rowsa, dca, 0)
    wait_g(rowsb, sgb)
    scat(rowsb, dca, 1)
    for j in range(4):
      @pl.when(rem_base + j < nw)
      def _():
        w = (base + rem_base + j) * _CH
        a = pltpu.async_copy(src_hbm.at[pl.ds(w, _CH)], sca.at[0], sidx)
        b = pltpu.async_copy(dst_hbm.at[pl.ds(w, _CH)], dca.at[0], sidx)
        a.wait()
        b.wait()
        gather(sca, 0, rowsa, sga)
        wait_g(rowsa, sga)
        scat(rowsa, dca, 0)

    plsc.subcore_barrier()
    _export_acc(out_ref, stga, stgb, z_hbm, acc, sga, sgb, c, s, n, zr, nchunk)

  return agg


@functools.lru_cache(maxsize=None)
def _make_sc_deg(n, e):
  assert e % _CH == 0
  nwin = e // _CH
  per, extra = _tile_windows(nwin)
  assert n % 16 == 0

  scratch = [
      pltpu.VMEM((per * _CH,), jnp.int32),
      pltpu.VMEM((_CH,), jnp.int32),
      pltpu.VMEM((n,), jnp.float32),
      pltpu.SemaphoreType.DMA,
  ]
  mesh = plsc.VectorSubcoreMesh(core_axis_name="c", subcore_axis_name="s",
                                num_cores=_NC, num_subcores=_NS)

  @functools.partial(
      pl.kernel, mesh=mesh,
      out_type=jax.ShapeDtypeStruct((_NW * n,), jnp.float32),
      scratch_types=scratch,
      compiler_params=pltpu.CompilerParams(needs_layout_passes=False))
  def deg(dst_hbm, out_ref, dall, dx, cnt, sem):
    c = lax.axis_index("c")
    s = lax.axis_index("s")
    wid = s * _NC + c
    base = wid * per + jnp.minimum(wid, extra)
    nw = per + jnp.where(wid < extra, 1, 0)

    def zstep(i, carry):
      cnt[pl.ds(i * 16, 16)] = jnp.zeros((16,), jnp.float32)
      return carry

    lax.fori_loop(0, n // 16, zstep, 0)

    pltpu.sync_copy(dst_hbm.at[pl.ds(base * _CH, per * _CH)], dall)

    @pl.when(nw > per)
    def _():
      pltpu.sync_copy(dst_hbm.at[pl.ds((base + per) * _CH, _CH)], dx)

    ones16 = jnp.ones((16,), jnp.float32)

    def hstep(i, carry):
      plsc.addupdate_scatter(cnt, [dall[pl.ds(i * 16, 16)]], ones16)
      return carry

    lax.fori_loop(0, per * _CH // 16, hstep, 0)

    @pl.when(nw > per)
    def _():
      def hstep2(i, carry):
        plsc.addupdate_scatter(cnt, [dx[pl.ds(i * 16, 16)]], ones16)
        return carry

      lax.fori_loop(0, _CH // 16, hstep2, 0)

    pltpu.sync_copy(cnt, out_ref.at[pl.ds(wid * n, n)])

  return deg


def _tc_dinv(cnt2d):
  n = cnt2d.shape[1]

  def body(cr, o):
    s = jnp.sum(cr[...], axis=0)
    o[...] = (1.0 / jnp.maximum(s, 1.0))[:, None]

  return pl.pallas_call(
      body,
      out_shape=jax.ShapeDtypeStruct((n, 1), jnp.float32),
  )(cnt2d)


def _tc_layer(agg_parts, dinv, w, b2d):
  n, h = agg_parts.shape[1], agg_parts.shape[2]
  blk = 2000 if n % 2000 == 0 else n
  grid = n // blk

  def body(ap, dp, wr, br, o):
    a = ap[0] + ap[1]
    z = lax.dot_general(a * dp[...], wr[...], (((1,), (0,)), ((), ())),
                        preferred_element_type=jnp.float32)
    o[...] = jnp.maximum(z + br[...], 0.0)

  return pl.pallas_call(
      body,
      grid=(grid,),
      in_specs=[
          pl.BlockSpec((2, blk, h), lambda i: (0, i, 0)),
          pl.BlockSpec((blk, 1), lambda i: (i, 0)),
          pl.BlockSpec((h, h), lambda i: (0, 0)),
          pl.BlockSpec((1, h), lambda i: (0, 0)),
      ],
      out_specs=pl.BlockSpec((blk, h), lambda i: (i, 0)),
      out_shape=jax.ShapeDtypeStruct((n, h), jnp.float32),
  )(agg_parts, dinv, w, b2d)


def _tc_layer_pool(agg_parts, dinv, w, b2d, batch2d, wc, bc2d):
  n, h = agg_parts.shape[1], agg_parts.shape[2]

  def body(ap, dp, wr, br, brow, wcr, bcr, o):
    a = ap[0] + ap[1]
    z = lax.dot_general(a * dp[...], wr[...], (((1,), (0,)), ((), ())),
                        preferred_element_type=jnp.float32)
    h3 = jnp.maximum(z + br[...], 0.0)
    gids = lax.broadcasted_iota(jnp.int32, (_G, n), 0)
    m = (brow[...] == gids).astype(jnp.float32)
    counts = jnp.sum(m, axis=1, keepdims=True)
    pooled = lax.dot_general(m, h3, (((1,), (0,)), ((), ())),
                             preferred_element_type=jnp.float32,
                             precision=lax.Precision.HIGHEST)
    pooled = pooled / jnp.maximum(counts, 1.0)
    o[...] = lax.dot_general(pooled, wcr[...], (((1,), (0,)), ((), ())),
                             preferred_element_type=jnp.float32) + bcr[...]

  return pl.pallas_call(
      body,
      out_shape=jax.ShapeDtypeStruct((_G, 1), jnp.float32),
  )(agg_parts, dinv, w, b2d, batch2d, wc, bc2d)


def kernel(x, edge_index, batch, W1, b1, W2, b2, W3, b3, Wc, bc):
  n, d = x.shape
  e = edge_index.shape[1]
  src = edge_index[0]
  dst = edge_index[1]
  z_rows = jnp.zeros((_ZCH, d), jnp.float32)

  agg = _make_sc_agg(n, e, d)
  deg = _make_sc_deg(n, e)

  dinv = _tc_dinv(deg(dst).reshape(_NW, n))
  a1 = agg(x, src, dst, z_rows)
  h1 = _tc_layer(a1.reshape(_NC, n, d), dinv, W1, b1.reshape(1, -1))
  a2 = agg(h1, src, dst, z_rows)
  h2 = _tc_layer(a2.reshape(_NC, n, d), dinv, W2, b2.reshape(1, -1))
  a3 = agg(h2, src, dst, z_rows)
  return _tc_layer_pool(a3.reshape(_NC, n, d), dinv, W3, b3.reshape(1, -1),
                        batch.reshape(1, -1), Wc, bc.reshape(1, 1))
